# Initial kernel scaffold; baseline (speedup 1.0000x reference)
import jax
import jax.numpy as jnp
from jax.experimental import pallas as pl


def kernel(x, weight, bias):
    raise NotImplementedError("write your pallas kernel here")


# speedup vs baseline: 1.2590x; 1.2590x over previous
import jax
import jax.numpy as jnp
from jax import lax
from jax.experimental import pallas as pl

TOP_K = 8
N_GROUPS = 64
NEG_INF = float("-inf")


def _gate_body(x_ref, w_ref, b_ref, idx_ref, wgt_ref):
    x_blk = x_ref[...]
    w = w_ref[...]
    logits = lax.dot_general(x_blk, w, (((1,), (1,)), ((), ())))
    logits = logits + b_ref[...]

    bt = logits.shape[0]
    e = jnp.exp(logits - jnp.max(logits, axis=1, keepdims=True))
    scores = e / jnp.sum(e, axis=1, keepdims=True)

    cols = lax.broadcasted_iota(jnp.int32, (bt, N_GROUPS), 1)
    vals = []
    s = scores
    for k in range(TOP_K):
        m = jnp.max(s, axis=1, keepdims=True)
        hit = s == m
        idx = jnp.min(jnp.where(hit, cols, N_GROUPS), axis=1, keepdims=True)
        idx_ref[:, k : k + 1] = idx
        vals.append(m)
        s = jnp.where(cols == idx, NEG_INF, s)
    v = jnp.concatenate(vals, axis=1)
    wgt_ref[...] = v / (jnp.sum(v, axis=1, keepdims=True) + 1e-20)


def kernel(x, weight, bias):
    bsz, seq_len, h = x.shape
    tokens = bsz * seq_len
    xf = x.reshape(tokens, h)
    b2 = bias.reshape(1, N_GROUPS)

    bt = 1024
    grid = (tokens // bt,)
    idx_out, wgt_out = pl.pallas_call(
        _gate_body,
        grid=grid,
        in_specs=[
            pl.BlockSpec((bt, h), lambda i: (i, 0)),
            pl.BlockSpec((N_GROUPS, h), lambda i: (0, 0)),
            pl.BlockSpec((1, N_GROUPS), lambda i: (0, 0)),
        ],
        out_specs=[
            pl.BlockSpec((bt, TOP_K), lambda i: (i, 0)),
            pl.BlockSpec((bt, TOP_K), lambda i: (i, 0)),
        ],
        out_shape=[
            jax.ShapeDtypeStruct((tokens, TOP_K), jnp.int32),
            jax.ShapeDtypeStruct((tokens, TOP_K), jnp.float32),
        ],
    )(xf, weight, b2)
    aux_loss = jnp.asarray(0.0, dtype=jnp.float32)
    return (idx_out, wgt_out, aux_loss)
